# baseline (speedup 1.0000x reference)
import functools

import jax
import jax.numpy as jnp
from jax.experimental import pallas as pl
from jax.experimental.pallas import tpu as pltpu


def _prep_body(wsc_ref, wih_ref, aT_ref):
    K = wsc_ref.shape[0]
    fourH = wih_ref.shape[0]
    for k in range(K):
        aT_ref[:, k * fourH:(k + 1) * fourH] = jax.lax.dot_general(
            wsc_ref[k], wih_ref[...],
            dimension_numbers=(((0,), (1,)), ((), ())),
            preferred_element_type=jnp.float32)


def _lstm_body(ev_ref, aT_ref, whhT_ref, wgT_ref, gamma_ref, beta_ref,
               out_ref, hst_ref, cst_ref, gx_ref, hs_ref, *, K, G, TC, H):
    fourH = 4 * H
    step = pl.program_id(1)

    @pl.when(step == 0)
    def _init():
        hst_ref[...] = jnp.zeros_like(hst_ref)
        cst_ref[...] = jnp.zeros_like(cst_ref)

    ev = jnp.swapaxes(ev_ref[...], 0, 1)
    ev2 = ev.reshape(TC * G, H).astype(jnp.bfloat16)
    for k in range(K):
        gx_ref[k] = jax.lax.dot_general(
            ev2, aT_ref[:, k * fourH:(k + 1) * fourH],
            dimension_numbers=(((1,), (0,)), ((), ())),
            preferred_element_type=jnp.float32).reshape(TC, G, fourH)

    whhT = whhT_ref[...]

    def body(t, carry):
        hprev, cprev = carry
        gxt = gx_ref[:, pl.ds(t, 1), :, :].reshape(K * G, fourH)
        gates = gxt + jax.lax.dot_general(
            hprev, whhT,
            dimension_numbers=(((1,), (0,)), ((), ())),
            preferred_element_type=jnp.float32)
        y = jnp.tanh(gates)
        i = 0.5 * y[:, 0:H] + 0.5
        f = 0.5 * y[:, H:2 * H] + 0.5
        g = y[:, 2 * H:3 * H]
        o = 0.5 * y[:, 3 * H:4 * H] + 0.5
        c = f * cprev + i * g
        hnew = o * jnp.tanh(c)
        hs_ref[pl.ds(t, 1)] = hnew.reshape(1, K * G, H)
        return hnew.astype(jnp.bfloat16), c

    hfin, cfin = jax.lax.fori_loop(
        0, TC, body, (hst_ref[...].astype(jnp.bfloat16), cst_ref[...]),
        unroll=8)
    hst_ref[...] = hfin.astype(jnp.float32)
    cst_ref[...] = cfin

    acc = jnp.zeros((TC * G, H), jnp.float32)
    for k in range(K):
        acc = acc + jax.lax.dot_general(
            hs_ref[:, k * G:(k + 1) * G, :].reshape(TC * G, H).astype(jnp.bfloat16),
            wgT_ref[k],
            dimension_numbers=(((1,), (0,)), ((), ())),
            preferred_element_type=jnp.float32)
    out = acc.reshape(TC, G, H) + ev
    mu = jnp.mean(out, axis=-1, keepdims=True)
    var = jnp.mean((out - mu) * (out - mu), axis=-1, keepdims=True)
    out = (out - mu) * jax.lax.rsqrt(var + 1e-5)
    out = (out * gamma_ref[...].reshape(1, 1, H)
           + beta_ref[...].reshape(1, 1, H))
    out_ref[...] = jnp.swapaxes(out, 0, 1)


def kernel(events, time, w, h, batch_id, lengths, batch_size,
           W_scatter, W_gather, W_ih, W_hh, ln_gamma, ln_beta):
    N, H = events.shape
    G = lengths.shape[0]
    L = N // G
    K = W_scatter.shape[0]
    fourH = W_ih.shape[0]
    TC = 128
    GH = G // 2

    Wsc = W_scatter.reshape(K, H, H)
    aT = pl.pallas_call(
        _prep_body,
        out_shape=jax.ShapeDtypeStruct((H, K * fourH), jnp.float32),
    )(Wsc, W_ih)

    col = jnp.arange(fourH)
    scale = jnp.where((col >= 2 * H) & (col < 3 * H), 1.0, 0.5)
    aT = (aT.reshape(H, K, fourH) * scale).reshape(H, K * fourH)

    ev3 = events.reshape(G, L, H)
    WgT = W_gather.reshape(K, H, H).transpose(0, 2, 1).astype(jnp.bfloat16)
    whhT = (W_hh.T * scale).astype(jnp.bfloat16)
    aT = aT.astype(jnp.bfloat16)
    out3 = pl.pallas_call(
        functools.partial(_lstm_body, K=K, G=GH, TC=TC, H=H),
        grid=(2, L // TC),
        in_specs=[
            pl.BlockSpec((GH, TC, H), lambda c, i: (c, i, 0)),
            pl.BlockSpec((H, K * fourH), lambda c, i: (0, 0)),
            pl.BlockSpec((H, fourH), lambda c, i: (0, 0)),
            pl.BlockSpec((K, H, H), lambda c, i: (0, 0, 0)),
            pl.BlockSpec((1, H), lambda c, i: (0, 0)),
            pl.BlockSpec((1, H), lambda c, i: (0, 0)),
        ],
        out_specs=pl.BlockSpec((GH, TC, H), lambda c, i: (c, i, 0)),
        out_shape=jax.ShapeDtypeStruct((G, L, H), jnp.float32),
        scratch_shapes=[
            pltpu.VMEM((K * GH, H), jnp.float32),
            pltpu.VMEM((K * GH, H), jnp.float32),
            pltpu.VMEM((K, TC, GH, fourH), jnp.float32),
            pltpu.VMEM((TC, K * GH, H), jnp.float32),
        ],
        compiler_params=pltpu.CompilerParams(
            dimension_semantics=("parallel", "arbitrary")),
    )(ev3, aT, whhT, WgT, ln_gamma.reshape(1, H), ln_beta.reshape(1, H))
    return out3.reshape(N, H)


# speedup vs baseline: 1.6136x; 1.6136x over previous
import functools

import jax
import jax.numpy as jnp
from jax.experimental import pallas as pl
from jax.experimental.pallas import tpu as pltpu


def _prep_body(wsc_ref, wih_ref, aT_ref):
    K = wsc_ref.shape[0]
    fourH = wih_ref.shape[0]
    for k in range(K):
        aT_ref[:, k * fourH:(k + 1) * fourH] = jax.lax.dot_general(
            wsc_ref[k], wih_ref[...],
            dimension_numbers=(((0,), (1,)), ((), ())),
            preferred_element_type=jnp.float32)


def _lstm_body(ev_ref, aT_ref, whhT_ref, wgT_ref, gamma_ref, beta_ref,
               out_ref, hst_ref, cst_ref, gx_ref, hs_ref, *, K, G, TC, H):
    fourH = 4 * H
    step = pl.program_id(0)

    @pl.when(step == 0)
    def _init():
        hst_ref[...] = jnp.zeros_like(hst_ref)
        cst_ref[...] = jnp.zeros_like(cst_ref)

    ev = jnp.swapaxes(ev_ref[...], 0, 1)
    ev2 = ev.reshape(TC * G, H).astype(jnp.bfloat16)
    for k in range(K):
        gx_ref[k] = jax.lax.dot_general(
            ev2, aT_ref[:, k * fourH:(k + 1) * fourH],
            dimension_numbers=(((1,), (0,)), ((), ())),
            preferred_element_type=jnp.float32).reshape(TC, G, fourH)

    whhT = whhT_ref[...]

    def body(t, carry):
        hprev, cprev = carry
        gxt = gx_ref[:, pl.ds(t, 1), :, :].reshape(K * G, fourH)
        gates = gxt + jax.lax.dot_general(
            hprev, whhT,
            dimension_numbers=(((1,), (0,)), ((), ())),
            preferred_element_type=jnp.float32)
        y = jnp.tanh(gates)
        i = 0.5 * y[:, 0:H] + 0.5
        f = 0.5 * y[:, H:2 * H] + 0.5
        g = y[:, 2 * H:3 * H]
        o = 0.5 * y[:, 3 * H:4 * H] + 0.5
        c = f * cprev + i * g
        hnew = o * jnp.tanh(c)
        hs_ref[pl.ds(t, 1)] = hnew.reshape(1, K * G, H)
        return hnew.astype(jnp.bfloat16), c

    hfin, cfin = jax.lax.fori_loop(
        0, TC, body, (hst_ref[...].astype(jnp.bfloat16), cst_ref[...]),
        unroll=8)
    hst_ref[...] = hfin.astype(jnp.float32)
    cst_ref[...] = cfin

    acc = jnp.zeros((TC * G, H), jnp.float32)
    for k in range(K):
        acc = acc + jax.lax.dot_general(
            hs_ref[:, k * G:(k + 1) * G, :].reshape(TC * G, H).astype(jnp.bfloat16),
            wgT_ref[k],
            dimension_numbers=(((1,), (0,)), ((), ())),
            preferred_element_type=jnp.float32)
    out = acc.reshape(TC, G, H) + ev
    mu = jnp.mean(out, axis=-1, keepdims=True)
    var = jnp.mean((out - mu) * (out - mu), axis=-1, keepdims=True)
    out = (out - mu) * jax.lax.rsqrt(var + 1e-5)
    out = (out * gamma_ref[...].reshape(1, 1, H)
           + beta_ref[...].reshape(1, 1, H))
    out_ref[...] = jnp.swapaxes(out, 0, 1)


def kernel(events, time, w, h, batch_id, lengths, batch_size,
           W_scatter, W_gather, W_ih, W_hh, ln_gamma, ln_beta):
    N, H = events.shape
    G = lengths.shape[0]
    L = N // G
    K = W_scatter.shape[0]
    fourH = W_ih.shape[0]
    TC = 128

    Wsc = W_scatter.reshape(K, H, H)
    aT = pl.pallas_call(
        _prep_body,
        out_shape=jax.ShapeDtypeStruct((H, K * fourH), jnp.float32),
    )(Wsc, W_ih)

    col = jnp.arange(fourH)
    scale = jnp.where((col >= 2 * H) & (col < 3 * H), 1.0, 0.5)
    aT = (aT.reshape(H, K, fourH) * scale).reshape(H, K * fourH)

    ev3 = events.reshape(G, L, H)
    WgT = W_gather.reshape(K, H, H).transpose(0, 2, 1).astype(jnp.bfloat16)
    whhT = (W_hh.T * scale).astype(jnp.bfloat16)
    aT = aT.astype(jnp.bfloat16)
    out3 = pl.pallas_call(
        functools.partial(_lstm_body, K=K, G=G, TC=TC, H=H),
        grid=(L // TC,),
        in_specs=[
            pl.BlockSpec((G, TC, H), lambda i: (0, i, 0)),
            pl.BlockSpec((H, K * fourH), lambda i: (0, 0)),
            pl.BlockSpec((H, fourH), lambda i: (0, 0)),
            pl.BlockSpec((K, H, H), lambda i: (0, 0, 0)),
            pl.BlockSpec((1, H), lambda i: (0, 0)),
            pl.BlockSpec((1, H), lambda i: (0, 0)),
        ],
        out_specs=pl.BlockSpec((G, TC, H), lambda i: (0, i, 0)),
        out_shape=jax.ShapeDtypeStruct((G, L, H), jnp.float32),
        scratch_shapes=[
            pltpu.VMEM((K * G, H), jnp.float32),
            pltpu.VMEM((K * G, H), jnp.float32),
            pltpu.VMEM((K, TC, G, fourH), jnp.float32),
            pltpu.VMEM((TC, K * G, H), jnp.float32),
        ],
        compiler_params=pltpu.CompilerParams(
            dimension_semantics=("arbitrary",)),
    )(ev3, aT, whhT, WgT, ln_gamma.reshape(1, H), ln_beta.reshape(1, H))
    return out3.reshape(N, H)
